# baseline (device time: 167656 ns/iter reference)
import jax
import jax.numpy as jnp
from jax import lax
from jax.experimental import pallas as pl
from jax.experimental.pallas import tpu as pltpu

N_Z = 4


def kernel(O, Wo):
    B, S, Hs, D = O.shape
    K = Hs * D
    N = Wo.shape[1]
    s_per = S // N_Z
    n_hops = N_Z - 1

    def body(
        x_ref, w_ref, out_ref, comm_ref, wb_ref, wstage_ref, obuf_ref,
        xbuf_ref, xs_ref, send_sems, recv_sems, out_sems, xdma_sems,
        wdma_sem,
    ):
        my_x = lax.axis_index("x")
        my_y = lax.axis_index("y")
        my_z = lax.axis_index("z")
        left = (my_z - 1) % N_Z
        right = (my_z + 1) % N_Z

        wdma = pltpu.make_async_copy(w_ref, wstage_ref, wdma_sem)
        wdma.start()

        barrier_sem = pltpu.get_barrier_semaphore()
        for nbr in (left, right):
            pl.semaphore_signal(
                barrier_sem, inc=1,
                device_id=(my_x, my_y, nbr),
                device_id_type=pl.DeviceIdType.MESH,
            )
        pl.semaphore_wait(barrier_sem, 2)
        wdma.wait()
        wb_ref[:, :] = wstage_ref[:, :].astype(jnp.bfloat16)

        def fetch_x(c, b, slot):
            dma = pltpu.make_async_copy(
                x_ref.at[b, pl.ds(c * s_per, s_per)],
                xbuf_ref.at[slot],
                xdma_sems.at[slot],
            )
            dma.start()
            return dma

        def chunk_f32(slot):
            for h8 in range(Hs):
                xs_ref[slot, :, pl.ds(h8 * D, D)] = (
                    xbuf_ref[slot, :, h8, :].astype(jnp.bfloat16)
                )
            return lax.dot_general(
                xs_ref[slot, :, :], wb_ref[:, :],
                (((1,), (0,)), ((), ())),
                preferred_element_type=jnp.float32,
            )

        rdmas = [
            [
                pltpu.make_async_remote_copy(
                    src_ref=comm_ref.at[h, b],
                    dst_ref=comm_ref.at[h + 1, b],
                    send_sem=send_sems.at[h, b],
                    recv_sem=recv_sems.at[h, b],
                    device_id=(my_x, my_y, right),
                    device_id_type=pl.DeviceIdType.MESH,
                )
                for b in range(B)
            ]
            for h in range(n_hops)
        ]

        c0 = (my_z - 1) % N_Z
        order = [(c0, b) for b in range(B)]
        for h in range(n_hops):
            c = (my_z - 2 - h) % N_Z
            order += [(c, b) for b in range(B)]

        fetch_x(order[0][0], order[0][1], 0)
        dmas = {}

        def ready_x(m):
            slot = m % 2
            if m + 1 < len(order):
                dmas[m + 1] = fetch_x(order[m + 1][0], order[m + 1][1],
                                      (m + 1) % 2)
            pltpu.make_async_copy(
                x_ref.at[order[m][1], pl.ds(order[m][0] * s_per, s_per)],
                xbuf_ref.at[slot],
                xdma_sems.at[slot],
            ).wait()
            return slot

        for b in range(B):
            slot = ready_x(b)
            comm_ref[0, b, :, :] = chunk_f32(slot).astype(jnp.bfloat16)
            rdmas[0][b].start()

        for h in range(n_hops):
            for b in range(B):
                m = (h + 1) * B + b
                slot = ready_x(m)
                if h < n_hops - 1:
                    t = chunk_f32(slot).astype(jnp.bfloat16)
                    rdmas[h][b].wait()
                    comm_ref[h + 1, b, :, :] = comm_ref[h + 1, b, :, :] + t
                    rdmas[h + 1][b].start()
                else:
                    t = chunk_f32(slot)
                    rdmas[h][b].wait()
                    obuf_ref[b, :, :] = (
                        t + comm_ref[h + 1, b, :, :].astype(jnp.float32)
                    )
                    pltpu.make_async_copy(
                        obuf_ref.at[b], out_ref.at[b], out_sems.at[b]
                    ).start()

        for b in range(B):
            pltpu.make_async_copy(
                obuf_ref.at[b], out_ref.at[b], out_sems.at[b]
            ).wait()

    return pl.pallas_call(
        body,
        out_shape=jax.ShapeDtypeStruct((B, s_per, N), jnp.float32),
        in_specs=[
            pl.BlockSpec(memory_space=pl.ANY),
            pl.BlockSpec(memory_space=pl.ANY),
        ],
        out_specs=pl.BlockSpec(memory_space=pl.ANY),
        scratch_shapes=[
            pltpu.VMEM((N_Z, B, s_per, N), jnp.bfloat16),
            pltpu.VMEM((K, N), jnp.bfloat16),
            pltpu.VMEM((K, N), jnp.float32),
            pltpu.VMEM((B, s_per, N), jnp.float32),
            pltpu.VMEM((2, s_per, Hs, D), jnp.float32),
            pltpu.VMEM((2, s_per, K), jnp.bfloat16),
            pltpu.SemaphoreType.DMA((N_Z - 1, B)),
            pltpu.SemaphoreType.DMA((N_Z - 1, B)),
            pltpu.SemaphoreType.DMA((B,)),
            pltpu.SemaphoreType.DMA((2,)),
            pltpu.SemaphoreType.DMA,
        ],
        compiler_params=pltpu.CompilerParams(
            collective_id=0, vmem_limit_bytes=100 * 1024 * 1024
        ),
    )(O, Wo)


# device time: 154777 ns/iter; 1.0832x vs baseline; 1.0832x over previous
import jax
import jax.numpy as jnp
from jax import lax
from jax.experimental import pallas as pl
from jax.experimental.pallas import tpu as pltpu

N_Z = 4


def kernel(O, Wo):
    B, S, Hs, D = O.shape
    K = Hs * D
    N = Wo.shape[1]
    s_per = S // N_Z
    n_hops = N_Z - 1

    x = O.transpose(0, 2, 3, 1).reshape(B, K, S)

    def body(
        x_ref, w_ref, out_ref, comm_ref, wb_ref, obuf_ref,
        send_sems, recv_sems, out_sems,
    ):
        my_x = lax.axis_index("x")
        my_y = lax.axis_index("y")
        my_z = lax.axis_index("z")
        left = (my_z - 1) % N_Z
        right = (my_z + 1) % N_Z

        barrier_sem = pltpu.get_barrier_semaphore()
        for nbr in (left, right):
            pl.semaphore_signal(
                barrier_sem, inc=1,
                device_id=(my_x, my_y, nbr),
                device_id_type=pl.DeviceIdType.MESH,
            )
        pl.semaphore_wait(barrier_sem, 2)

        wb_ref[:, :] = w_ref[:, :].astype(jnp.bfloat16)

        def chunk_f32(c, b):
            xs = x_ref[b, :, pl.ds(c * s_per, s_per)].astype(jnp.bfloat16)
            return lax.dot_general(
                xs, wb_ref[:, :],
                (((0,), (0,)), ((), ())),
                preferred_element_type=jnp.float32,
            )

        rdmas = [
            [
                pltpu.make_async_remote_copy(
                    src_ref=comm_ref.at[h, b],
                    dst_ref=comm_ref.at[h + 1, b],
                    send_sem=send_sems.at[h, b],
                    recv_sem=recv_sems.at[h, b],
                    device_id=(my_x, my_y, right),
                    device_id_type=pl.DeviceIdType.MESH,
                )
                for b in range(B)
            ]
            for h in range(n_hops)
        ]

        c0 = (my_z - 1) % N_Z
        for b in range(B):
            comm_ref[0, b, :, :] = chunk_f32(c0, b).astype(jnp.bfloat16)
            rdmas[0][b].start()

        for h in range(n_hops):
            c = (my_z - 2 - h) % N_Z
            for b in range(B):
                if h < n_hops - 1:
                    t = chunk_f32(c, b).astype(jnp.bfloat16)
                    rdmas[h][b].wait()
                    comm_ref[h + 1, b, :, :] = comm_ref[h + 1, b, :, :] + t
                    rdmas[h + 1][b].start()
                else:
                    t = chunk_f32(c, b)
                    rdmas[h][b].wait()
                    obuf_ref[b, :, :] = (
                        t + comm_ref[h + 1, b, :, :].astype(jnp.float32)
                    )
                    pltpu.make_async_copy(
                        obuf_ref.at[b], out_ref.at[b], out_sems.at[b]
                    ).start()

        for b in range(B):
            pltpu.make_async_copy(
                obuf_ref.at[b], out_ref.at[b], out_sems.at[b]
            ).wait()

    return pl.pallas_call(
        body,
        out_shape=jax.ShapeDtypeStruct((B, s_per, N), jnp.float32),
        in_specs=[
            pl.BlockSpec(memory_space=pltpu.VMEM),
            pl.BlockSpec(memory_space=pltpu.VMEM),
        ],
        out_specs=pl.BlockSpec(memory_space=pl.ANY),
        scratch_shapes=[
            pltpu.VMEM((N_Z, B, s_per, N), jnp.bfloat16),
            pltpu.VMEM((K, N), jnp.bfloat16),
            pltpu.VMEM((B, s_per, N), jnp.float32),
            pltpu.SemaphoreType.DMA((N_Z - 1, B)),
            pltpu.SemaphoreType.DMA((N_Z - 1, B)),
            pltpu.SemaphoreType.DMA((B,)),
        ],
        compiler_params=pltpu.CompilerParams(
            collective_id=0, vmem_limit_bytes=100 * 1024 * 1024
        ),
    )(x, Wo)


# device time: 153662 ns/iter; 1.0911x vs baseline; 1.0073x over previous
import jax
import jax.numpy as jnp
from jax import lax
from jax.experimental import pallas as pl
from jax.experimental.pallas import tpu as pltpu

N_Z = 4


def kernel(O, Wo):
    B, S, Hs, D = O.shape
    K = Hs * D
    N = Wo.shape[1]
    s_per = S // N_Z
    n_hops = N_Z - 1

    x = O.transpose(0, 2, 3, 1).reshape(B, K, S)

    def body(
        x_ref, w_ref, out_ref, comm_ref, wb_ref, obuf_ref,
        send_sems, recv_sems, last_send_sems, last_recv_sems, out_sems,
    ):
        my_x = lax.axis_index("x")
        my_y = lax.axis_index("y")
        my_z = lax.axis_index("z")
        left = (my_z - 1) % N_Z
        right = (my_z + 1) % N_Z

        barrier_sem = pltpu.get_barrier_semaphore()
        for nbr in (left, right):
            pl.semaphore_signal(
                barrier_sem, inc=1,
                device_id=(my_x, my_y, nbr),
                device_id_type=pl.DeviceIdType.MESH,
            )
        pl.semaphore_wait(barrier_sem, 2)

        wb_ref[:, :] = w_ref[:, :].astype(jnp.bfloat16)

        def chunk_f32(c, b):
            xs = x_ref[b, :, pl.ds(c * s_per, s_per)].astype(jnp.bfloat16)
            return lax.dot_general(
                xs, wb_ref[:, :],
                (((0,), (0,)), ((), ())),
                preferred_element_type=jnp.float32,
            )

        rdmas = [
            [
                pltpu.make_async_remote_copy(
                    src_ref=comm_ref.at[h, b],
                    dst_ref=comm_ref.at[h + 1, b],
                    send_sem=send_sems.at[h, b],
                    recv_sem=recv_sems.at[h, b],
                    device_id=(my_x, my_y, right),
                    device_id_type=pl.DeviceIdType.MESH,
                )
                for b in range(B)
            ]
            for h in range(n_hops - 1)
        ]
        half = s_per // 2
        last_rdmas = [
            [
                pltpu.make_async_remote_copy(
                    src_ref=comm_ref.at[n_hops - 1, b, pl.ds(i * half, half)],
                    dst_ref=comm_ref.at[n_hops, b, pl.ds(i * half, half)],
                    send_sem=last_send_sems.at[b, i],
                    recv_sem=last_recv_sems.at[b, i],
                    device_id=(my_x, my_y, right),
                    device_id_type=pl.DeviceIdType.MESH,
                )
                for i in range(2)
            ]
            for b in range(B)
        ]

        c0 = (my_z - 1) % N_Z
        for b in range(B):
            comm_ref[0, b, :, :] = chunk_f32(c0, b).astype(jnp.bfloat16)
            rdmas[0][b].start()

        for h in range(n_hops - 1):
            c = (my_z - 2 - h) % N_Z
            for b in range(B):
                t = chunk_f32(c, b).astype(jnp.bfloat16)
                rdmas[h][b].wait()
                comm_ref[h + 1, b, :, :] = comm_ref[h + 1, b, :, :] + t
                if h < n_hops - 2:
                    rdmas[h + 1][b].start()
                else:
                    last_rdmas[b][0].start()
                    last_rdmas[b][1].start()

        for b in range(B):
            t = chunk_f32(my_z, b)
            for i in range(2):
                sl = pl.ds(i * half, half)
                last_rdmas[b][i].wait()
                obuf_ref[b, sl, :] = (
                    t[i * half:(i + 1) * half, :]
                    + comm_ref[n_hops, b, sl, :].astype(jnp.float32)
                )
                pltpu.make_async_copy(
                    obuf_ref.at[b, sl], out_ref.at[b, sl], out_sems.at[b, i]
                ).start()

        for b in range(B):
            for i in range(2):
                sl = pl.ds(i * half, half)
                pltpu.make_async_copy(
                    obuf_ref.at[b, sl], out_ref.at[b, sl], out_sems.at[b, i]
                ).wait()

    return pl.pallas_call(
        body,
        out_shape=jax.ShapeDtypeStruct((B, s_per, N), jnp.float32),
        in_specs=[
            pl.BlockSpec(memory_space=pltpu.VMEM),
            pl.BlockSpec(memory_space=pltpu.VMEM),
        ],
        out_specs=pl.BlockSpec(memory_space=pl.ANY),
        scratch_shapes=[
            pltpu.VMEM((N_Z, B, s_per, N), jnp.bfloat16),
            pltpu.VMEM((K, N), jnp.bfloat16),
            pltpu.VMEM((B, s_per, N), jnp.float32),
            pltpu.SemaphoreType.DMA((N_Z - 2, B)),
            pltpu.SemaphoreType.DMA((N_Z - 2, B)),
            pltpu.SemaphoreType.DMA((B, 2)),
            pltpu.SemaphoreType.DMA((B, 2)),
            pltpu.SemaphoreType.DMA((B, 2)),
        ],
        compiler_params=pltpu.CompilerParams(
            collective_id=0, vmem_limit_bytes=100 * 1024 * 1024
        ),
    )(x, Wo)
